# Initial kernel scaffold; baseline (speedup 1.0000x reference)
import jax
import jax.numpy as jnp
from jax.experimental import pallas as pl


def kernel(src, Wq, bq, Wk, bk, Wv, bv, Wo, bo, Wg, bg, W1, b1, W2, b2, g1, beta1, g2, beta2):
    raise NotImplementedError("write your pallas kernel here")


# speedup vs baseline: 1.2647x; 1.2647x over previous
import functools

import jax
import jax.numpy as jnp
from jax import lax
from jax.experimental import pallas as pl
from jax.experimental.pallas import tpu as pltpu

EPS = 1e-5
HEAD_DIM = 64
LANE = 128


def _proj_kernel(x_ref, w_ref, b_ref, o_ref):
    o_ref[...] = (
        jnp.dot(x_ref[...], w_ref[...], preferred_element_type=jnp.float32)
        + b_ref[...]
    )


def _qkv_proj(x, w, b, bm):
    M, K = x.shape
    Nn = w.shape[1]
    return pl.pallas_call(
        _proj_kernel,
        grid=(M // bm,),
        in_specs=[
            pl.BlockSpec((bm, K), lambda i: (i, 0)),
            pl.BlockSpec((K, Nn), lambda i: (0, 0)),
            pl.BlockSpec((1, Nn), lambda i: (0, 0)),
        ],
        out_specs=pl.BlockSpec((bm, Nn), lambda i: (i, 0)),
        out_shape=jax.ShapeDtypeStruct((M, Nn), jnp.float32),
        compiler_params=pltpu.CompilerParams(
            dimension_semantics=("arbitrary",)
        ),
    )(x, w, b.reshape(1, Nn))


def _attn_kernel(q_ref, k_ref, v_ref, o_ref, *, scale):
    for hh in range(2):
        sl = slice(hh * HEAD_DIM, (hh + 1) * HEAD_DIM)
        q = q_ref[0, :, sl]
        k = k_ref[0, :, sl]
        v = v_ref[0, :, sl]
        s = lax.dot_general(
            q, k, (((1,), (1,)), ((), ())), preferred_element_type=jnp.float32
        ) * scale
        m = jnp.max(s, axis=-1, keepdims=True)
        p = jnp.exp(s - m)
        p = p / jnp.sum(p, axis=-1, keepdims=True)
        o_ref[0, :, sl] = jnp.dot(p, v, preferred_element_type=jnp.float32)


def _attention(qkv, B, T, D, bq):
    scale = HEAD_DIM ** (-0.5)
    HP = D // LANE
    return pl.pallas_call(
        functools.partial(_attn_kernel, scale=scale),
        grid=(B * HP, T // bq),
        in_specs=[
            pl.BlockSpec(
                (1, bq, LANE), lambda bp, iq: (bp // HP, iq, bp % HP)
            ),
            pl.BlockSpec(
                (1, T, LANE), lambda bp, iq: (bp // HP, 0, HP + bp % HP)
            ),
            pl.BlockSpec(
                (1, T, LANE), lambda bp, iq: (bp // HP, 0, 2 * HP + bp % HP)
            ),
        ],
        out_specs=pl.BlockSpec(
            (1, bq, LANE), lambda bp, iq: (bp // HP, iq, bp % HP)
        ),
        out_shape=jax.ShapeDtypeStruct((B, T, D), jnp.float32),
        compiler_params=pltpu.CompilerParams(
            dimension_semantics=("arbitrary", "arbitrary")
        ),
    )(qkv, qkv, qkv)


def _postattn_kernel(
    o_ref, wo_ref, bo_ref, src_ref, g_ref, be_ref, wg_ref, bg_ref,
    x_ref, w_ref, i_ref,
):
    attn = (
        jnp.dot(o_ref[...], wo_ref[...], preferred_element_type=jnp.float32)
        + bo_ref[...]
    )
    x = src_ref[...] + attn
    mu = jnp.mean(x, axis=-1, keepdims=True)
    var = jnp.mean((x - mu) ** 2, axis=-1, keepdims=True)
    xn = (x - mu) * lax.rsqrt(var + EPS) * g_ref[...] + be_ref[...]
    x_ref[...] = xn
    logits = (
        jnp.dot(xn, wg_ref[...], preferred_element_type=jnp.float32)
        + bg_ref[...]
    )
    mx = jnp.max(logits, axis=-1, keepdims=True)
    e = jnp.exp(logits - mx)
    probs = e / jnp.sum(e, axis=-1, keepdims=True)
    idx = lax.broadcasted_iota(jnp.int32, probs.shape, 1)
    w0 = jnp.max(probs, axis=-1, keepdims=True)
    i0 = jnp.min(jnp.where(probs == w0, idx, LANE), axis=-1, keepdims=True)
    probs2 = jnp.where(idx == i0, -1.0, probs)
    w1 = jnp.max(probs2, axis=-1, keepdims=True)
    i1 = jnp.min(jnp.where(probs2 == w1, idx, LANE), axis=-1, keepdims=True)
    w_ref[...] = jnp.where(idx == 0, w0, jnp.where(idx == 1, w1, 0.0))
    i_ref[...] = jnp.where(idx == 0, i0, jnp.where(idx == 1, i1, 0))


def _postattn(o2d, Wo, bo, src2d, g1, beta1, wg_pad, bg_pad, bm):
    N, D = o2d.shape
    return pl.pallas_call(
        _postattn_kernel,
        grid=(N // bm,),
        in_specs=[
            pl.BlockSpec((bm, D), lambda i: (i, 0)),
            pl.BlockSpec((D, D), lambda i: (0, 0)),
            pl.BlockSpec((1, D), lambda i: (0, 0)),
            pl.BlockSpec((bm, D), lambda i: (i, 0)),
            pl.BlockSpec((1, D), lambda i: (0, 0)),
            pl.BlockSpec((1, D), lambda i: (0, 0)),
            pl.BlockSpec((D, LANE), lambda i: (0, 0)),
            pl.BlockSpec((1, LANE), lambda i: (0, 0)),
        ],
        out_specs=[
            pl.BlockSpec((bm, D), lambda i: (i, 0)),
            pl.BlockSpec((bm, LANE), lambda i: (i, 0)),
            pl.BlockSpec((bm, LANE), lambda i: (i, 0)),
        ],
        out_shape=[
            jax.ShapeDtypeStruct((N, D), jnp.float32),
            jax.ShapeDtypeStruct((N, LANE), jnp.float32),
            jax.ShapeDtypeStruct((N, LANE), jnp.int32),
        ],
        compiler_params=pltpu.CompilerParams(
            dimension_semantics=("arbitrary",)
        ),
    )(
        o2d, Wo, bo.reshape(1, D), src2d,
        g1.reshape(1, D), beta1.reshape(1, D), wg_pad, bg_pad,
    )


def _moe_ffn_kernel(be_ref, xs_ref, w1_ref, b1_ref, w2_ref, b2_ref, ys_ref):
    del be_ref
    h = (
        jnp.dot(xs_ref[...], w1_ref[0], preferred_element_type=jnp.float32)
        + b1_ref[0, 0]
    )
    h = jnp.maximum(h, 0.0)
    ys_ref[...] = (
        jnp.dot(h, w2_ref[0], preferred_element_type=jnp.float32)
        + b2_ref[0, 0]
    )


def _moe_ffn(xs, W1, b1, W2, b2, block_expert, bm):
    P, D = xs.shape
    FF = W1.shape[2]
    NB = P // bm
    grid_spec = pltpu.PrefetchScalarGridSpec(
        num_scalar_prefetch=1,
        grid=(NB,),
        in_specs=[
            pl.BlockSpec((bm, D), lambda i, be: (i, 0)),
            pl.BlockSpec((1, D, FF), lambda i, be: (be[i], 0, 0)),
            pl.BlockSpec((1, 1, FF), lambda i, be: (be[i], 0, 0)),
            pl.BlockSpec((1, FF, D), lambda i, be: (be[i], 0, 0)),
            pl.BlockSpec((1, 1, D), lambda i, be: (be[i], 0, 0)),
        ],
        out_specs=pl.BlockSpec((bm, D), lambda i, be: (i, 0)),
    )
    return pl.pallas_call(
        _moe_ffn_kernel,
        grid_spec=grid_spec,
        out_shape=jax.ShapeDtypeStruct((P, D), jnp.float32),
        compiler_params=pltpu.CompilerParams(
            dimension_semantics=("arbitrary",)
        ),
    )(
        block_expert, xs, W1,
        b1.reshape(b1.shape[0], 1, FF), W2, b2.reshape(b2.shape[0], 1, D),
    )


def _combine_kernel(x_ref, y0_ref, y1_ref, w_ref, g_ref, be_ref, o_ref):
    w0 = w_ref[:, 0:1]
    w1 = w_ref[:, 1:2]
    s = x_ref[...] + w0 * y0_ref[...] + w1 * y1_ref[...]
    mu = jnp.mean(s, axis=-1, keepdims=True)
    var = jnp.mean((s - mu) ** 2, axis=-1, keepdims=True)
    o_ref[...] = (s - mu) * lax.rsqrt(var + EPS) * g_ref[...] + be_ref[...]


def _combine(x, y0, y1, wout, g2, beta2, bm):
    N, D = x.shape
    return pl.pallas_call(
        _combine_kernel,
        grid=(N // bm,),
        in_specs=[
            pl.BlockSpec((bm, D), lambda i: (i, 0)),
            pl.BlockSpec((bm, D), lambda i: (i, 0)),
            pl.BlockSpec((bm, D), lambda i: (i, 0)),
            pl.BlockSpec((bm, LANE), lambda i: (i, 0)),
            pl.BlockSpec((1, D), lambda i: (0, 0)),
            pl.BlockSpec((1, D), lambda i: (0, 0)),
        ],
        out_specs=pl.BlockSpec((bm, D), lambda i: (i, 0)),
        out_shape=jax.ShapeDtypeStruct((N, D), jnp.float32),
        compiler_params=pltpu.CompilerParams(
            dimension_semantics=("arbitrary",)
        ),
    )(x, y0, y1, wout, g2.reshape(1, D), beta2.reshape(1, D))


def kernel(src, Wq, bq, Wk, bk, Wv, bv, Wo, bo, Wg, bg,
           W1, b1, W2, b2, g1, beta1, g2, beta2):
    B, T, D = src.shape
    E = Wg.shape[1]
    FF = W1.shape[2]
    H = D // HEAD_DIM
    N = B * T
    bm_moe = 128
    P = 2 * N + E * bm_moe
    NB = P // bm_moe

    x2d = src.reshape(N, D)

    Wqkv = jnp.concatenate([Wq, Wk, Wv], axis=1)
    bqkv = jnp.concatenate([bq, bk, bv])
    qkv = _qkv_proj(x2d, Wqkv, bqkv, bm=min(256, N))
    qkv3 = qkv.reshape(B, T, 3 * D)
    o = _attention(qkv3, B, T, D, bq=min(512, T))
    o2d = o.reshape(N, D)

    wg_pad = jnp.zeros((D, LANE), jnp.float32).at[:, :E].set(Wg)
    bg_pad = jnp.full((1, LANE), -1e30, jnp.float32).at[0, :E].set(bg)
    x, wout, iout = _postattn(
        o2d, Wo, bo, x2d, g1, beta1, wg_pad, bg_pad, bm=min(256, N)
    )

    e_all = jnp.concatenate([iout[:, 0], iout[:, 1]])
    onehot = (e_all[:, None] == jnp.arange(E)[None, :]).astype(jnp.int32)
    counts = jnp.sum(onehot, axis=0)
    ca = ((counts + bm_moe - 1) // bm_moe) * bm_moe
    starts = jnp.concatenate(
        [jnp.zeros((1,), ca.dtype), jnp.cumsum(ca)[:-1]]
    )
    rank = jnp.cumsum(onehot, axis=0) - 1
    rank_j = jnp.take_along_axis(rank, e_all[:, None], axis=1)[:, 0]
    p = (starts[e_all] + rank_j).astype(jnp.int32)
    tok = jnp.tile(jnp.arange(N, dtype=jnp.int32), 2)
    sorted_tok = jnp.zeros((P,), jnp.int32).at[p].set(tok)
    rb = jnp.arange(NB, dtype=starts.dtype) * bm_moe
    block_expert = jnp.clip(
        jnp.searchsorted(starts, rb, side="right") - 1, 0, E - 1
    ).astype(jnp.int32)

    xs = jnp.take(x, sorted_tok, axis=0)
    ys = _moe_ffn(xs, W1, b1, W2, b2, block_expert, bm=bm_moe)
    y0 = jnp.take(ys, p[:N], axis=0)
    y1 = jnp.take(ys, p[N:], axis=0)

    out2d = _combine(x, y0, y1, wout, g2, beta2, bm=min(256, N))
    return out2d.reshape(B, T, D)
